# baseline (speedup 1.0000x reference)
import functools

import jax
import jax.numpy as jnp
from jax import lax
from jax.experimental import pallas as pl
from jax.experimental.pallas import tpu as pltpu
from jax.experimental.pallas import tpu_sc as plsc

N = 10000
D = 128
HD = D // 2
E = 320000

NC = 2
NS = 16
CHUNK = 128
T = 160
E_PAD = NS * T * CHUNK
N_PAD = 10240
RPT = N_PAD // NS
ZR = 128


NBUF = 4
IB = 40


LT_ROWS = N - (NS - 1) * RPT


def _spmm_body(z_hbm, src_hbm, dst_hbm, b_hbm, out_hbm,
               src_v, dst_v, rows, bias_v, ztab, acc, gsem, ssem):
    c = lax.axis_index("c")
    s = lax.axis_index("s")

    @pl.when(s < NS - 1)
    def _():
        pltpu.sync_copy(z_hbm.at[pl.ds(s * RPT, RPT), pl.ds(c * HD, HD)],
                        ztab.at[pl.ds(s * RPT, RPT)])

    @pl.when(s == NS - 1)
    def _():
        pltpu.sync_copy(
            z_hbm.at[pl.ds((NS - 1) * RPT, LT_ROWS), pl.ds(c * HD, HD)],
            ztab.at[pl.ds((NS - 1) * RPT, LT_ROWS)])

    pltpu.sync_copy(b_hbm.at[c], bias_v)
    bvals = [bias_v[pl.ds(k * 16, 16)] for k in range(HD // 16)]

    @pl.loop(0, ZR)
    def _bias_row(r):
        for k in range(HD // 16):
            rows[0][r, pl.ds(k * 16, 16)] = bvals[k]

    for k in range(RPT // ZR):
        pltpu.sync_copy(rows[0], acc.at[pl.ds(s * RPT + k * ZR, ZR)])
    plsc.subcore_barrier()

    @pl.loop(0, T // IB)
    def _blk(blk):
        pltpu.sync_copy(src_hbm.at[pl.ds(s * T + blk * IB, IB)], src_v)
        pltpu.sync_copy(dst_hbm.at[pl.ds(s * T + blk * IB, IB)], dst_v)

        for b in range(NBUF):
            pltpu.async_copy(ztab.at[src_v.at[b]], rows[b], gsem[b])

        @pl.loop(0, IB, step=NBUF)
        def _chunk(j):
            for b in range(NBUF):
                pltpu.make_async_copy(ztab.at[src_v.at[j + b]],
                                      rows[b], gsem[b]).wait()
                pltpu.async_copy(rows[b], acc.at[dst_v.at[j + b]],
                                 ssem[b], add=True)
            for b in range(NBUF):
                pltpu.make_async_copy(rows[b], acc.at[dst_v.at[j + b]],
                                      ssem[b]).wait()

                @pl.when(j + NBUF + b < IB)
                def _():
                    pltpu.async_copy(ztab.at[src_v.at[j + NBUF + b]],
                                     rows[b], gsem[b])

    plsc.subcore_barrier()

    @pl.when(s < NS - 1)
    def _():
        pltpu.sync_copy(acc.at[pl.ds(s * RPT, RPT)],
                        out_hbm.at[pl.ds(s * RPT, RPT), pl.ds(c * HD, HD)])

    @pl.when(s == NS - 1)
    def _():
        pltpu.sync_copy(
            acc.at[pl.ds((NS - 1) * RPT, LT_ROWS)],
            out_hbm.at[pl.ds((NS - 1) * RPT, LT_ROWS), pl.ds(c * HD, HD)])


_spmm = functools.partial(
    pl.kernel,
    out_type=jax.ShapeDtypeStruct((N, D), jnp.float32),
    mesh=plsc.VectorSubcoreMesh(core_axis_name="c", subcore_axis_name="s",
                                num_cores=NC, num_subcores=NS),
    compiler_params=pltpu.CompilerParams(use_tc_tiling_on_sc=False),
    scratch_types=[
        pltpu.VMEM((IB, CHUNK), jnp.int32),
        pltpu.VMEM((IB, CHUNK), jnp.int32),
        tuple(pltpu.VMEM((CHUNK, HD), jnp.float32)
              for _ in range(NBUF)),
        pltpu.VMEM((HD,), jnp.float32),
        pltpu.VMEM_SHARED((N_PAD, HD), jnp.float32),
        pltpu.VMEM_SHARED((N_PAD, HD), jnp.float32),
        tuple(pltpu.SemaphoreType.DMA for _ in range(NBUF)),
        tuple(pltpu.SemaphoreType.DMA for _ in range(NBUF)),
    ],
)(_spmm_body)


BR = 1000


def _dense_body(p_ref, w1_ref, b_ref, w2_ref, o_ref):
    t = jnp.dot(p_ref[...], w1_ref[...],
                preferred_element_type=jnp.float32) + b_ref[...]
    t = jnp.maximum(t, 0.0)
    o_ref[...] = jnp.dot(t, w2_ref[...], preferred_element_type=jnp.float32)


_dense = pl.pallas_call(
    _dense_body,
    grid=(N // BR,),
    in_specs=[
        pl.BlockSpec((BR, D), lambda i: (i, 0)),
        pl.BlockSpec((D, D), lambda i: (0, 0)),
        pl.BlockSpec((1, D), lambda i: (0, 0)),
        pl.BlockSpec((D, D), lambda i: (0, 0)),
    ],
    out_specs=pl.BlockSpec((BR, D), lambda i: (i, 0)),
    out_shape=jax.ShapeDtypeStruct((N, D), jnp.float32),
)


def kernel(x, edge_index, W1, b1, W2, b2):
    src = edge_index[0]
    dst = edge_index[1]
    pad = E_PAD - E
    src_i = jnp.concatenate(
        [src, jnp.zeros((pad,), jnp.int32)]).reshape(NS * T, CHUNK)
    dst_i = jnp.concatenate(
        [dst, jnp.full((pad,), N, jnp.int32)]).reshape(NS * T, CHUNK)
    zb = jnp.zeros((NC, HD), jnp.float32)
    b1r = b1.reshape(1, D)
    b2s = b2.reshape(NC, HD)

    ax = _spmm(x, src_i, dst_i, zb)
    z2 = _dense(ax, W1, b1r, W2)
    return _spmm(z2, src_i, dst_i, b2s)


# speedup vs baseline: 1.0203x; 1.0203x over previous
import functools

import jax
import jax.numpy as jnp
from jax import lax
from jax.experimental import pallas as pl
from jax.experimental.pallas import tpu as pltpu
from jax.experimental.pallas import tpu_sc as plsc

N = 10000
D = 128
HD = D // 2
E = 320000

NC = 2
NS = 16
CHUNK = 128
T = 160
E_PAD = NS * T * CHUNK
N_PAD = 10240
RPT = N_PAD // NS
ZR = 128


NBUF = 4
IB = 32


LT_ROWS = N - (NS - 1) * RPT


def _spmm_body(z_hbm, src_hbm, dst_hbm, b_hbm, out_hbm,
               src_v, dst_v, rows, bias_v, ztab, acc, gsem, ssem):
    c = lax.axis_index("c")
    s = lax.axis_index("s")

    @pl.when(s < NS - 1)
    def _():
        pltpu.async_copy(z_hbm.at[pl.ds(s * RPT, RPT), pl.ds(c * HD, HD)],
                         ztab.at[pl.ds(s * RPT, RPT)], gsem[0])

    @pl.when(s == NS - 1)
    def _():
        pltpu.async_copy(
            z_hbm.at[pl.ds((NS - 1) * RPT, LT_ROWS), pl.ds(c * HD, HD)],
            ztab.at[pl.ds((NS - 1) * RPT, LT_ROWS)], gsem[0])

    pltpu.sync_copy(b_hbm.at[c], bias_v)
    bvals = [bias_v[pl.ds(k * 16, 16)] for k in range(HD // 16)]

    @pl.loop(0, ZR)
    def _bias_row(r):
        for k in range(HD // 16):
            rows[0][r, pl.ds(k * 16, 16)] = bvals[k]

    for k in range(RPT // ZR):
        pltpu.sync_copy(rows[0], acc.at[pl.ds(s * RPT + k * ZR, ZR)])

    @pl.when(s < NS - 1)
    def _():
        pltpu.make_async_copy(
            z_hbm.at[pl.ds(s * RPT, RPT), pl.ds(c * HD, HD)],
            ztab.at[pl.ds(s * RPT, RPT)], gsem[0]).wait()

    @pl.when(s == NS - 1)
    def _():
        pltpu.make_async_copy(
            z_hbm.at[pl.ds((NS - 1) * RPT, LT_ROWS), pl.ds(c * HD, HD)],
            ztab.at[pl.ds((NS - 1) * RPT, LT_ROWS)], gsem[0]).wait()

    plsc.subcore_barrier()

    @pl.loop(0, T // IB)
    def _blk(blk):
        pltpu.sync_copy(src_hbm.at[pl.ds(s * T + blk * IB, IB)], src_v)
        pltpu.sync_copy(dst_hbm.at[pl.ds(s * T + blk * IB, IB)], dst_v)

        for b in range(NBUF):
            pltpu.async_copy(ztab.at[src_v.at[b]], rows[b], gsem[b])

        @pl.loop(0, IB, step=NBUF)
        def _chunk(j):
            for b in range(NBUF):
                pltpu.make_async_copy(ztab.at[src_v.at[j + b]],
                                      rows[b], gsem[b]).wait()
                pltpu.async_copy(rows[b], acc.at[dst_v.at[j + b]],
                                 ssem[b], add=True)
            for b in range(NBUF):
                pltpu.make_async_copy(rows[b], acc.at[dst_v.at[j + b]],
                                      ssem[b]).wait()

                @pl.when(j + NBUF + b < IB)
                def _():
                    pltpu.async_copy(ztab.at[src_v.at[j + NBUF + b]],
                                     rows[b], gsem[b])

    plsc.subcore_barrier()

    @pl.when(s < NS - 1)
    def _():
        pltpu.sync_copy(acc.at[pl.ds(s * RPT, RPT)],
                        out_hbm.at[pl.ds(s * RPT, RPT), pl.ds(c * HD, HD)])

    @pl.when(s == NS - 1)
    def _():
        pltpu.sync_copy(
            acc.at[pl.ds((NS - 1) * RPT, LT_ROWS)],
            out_hbm.at[pl.ds((NS - 1) * RPT, LT_ROWS), pl.ds(c * HD, HD)])


_spmm = functools.partial(
    pl.kernel,
    out_type=jax.ShapeDtypeStruct((N, D), jnp.float32),
    mesh=plsc.VectorSubcoreMesh(core_axis_name="c", subcore_axis_name="s",
                                num_cores=NC, num_subcores=NS),
    compiler_params=pltpu.CompilerParams(use_tc_tiling_on_sc=False),
    scratch_types=[
        pltpu.VMEM((IB, CHUNK), jnp.int32),
        pltpu.VMEM((IB, CHUNK), jnp.int32),
        tuple(pltpu.VMEM((CHUNK, HD), jnp.float32)
              for _ in range(NBUF)),
        pltpu.VMEM((HD,), jnp.float32),
        pltpu.VMEM_SHARED((N_PAD, HD), jnp.float32),
        pltpu.VMEM_SHARED((N_PAD, HD), jnp.float32),
        tuple(pltpu.SemaphoreType.DMA for _ in range(NBUF)),
        tuple(pltpu.SemaphoreType.DMA for _ in range(NBUF)),
    ],
)(_spmm_body)


BR = 1000


def _dense_body(p_ref, w1_ref, b_ref, w2_ref, o_ref):
    t = jnp.dot(p_ref[...], w1_ref[...],
                preferred_element_type=jnp.float32) + b_ref[...]
    t = jnp.maximum(t, 0.0)
    o_ref[...] = jnp.dot(t, w2_ref[...], preferred_element_type=jnp.float32)


_dense = pl.pallas_call(
    _dense_body,
    grid=(N // BR,),
    in_specs=[
        pl.BlockSpec((BR, D), lambda i: (i, 0)),
        pl.BlockSpec((D, D), lambda i: (0, 0)),
        pl.BlockSpec((1, D), lambda i: (0, 0)),
        pl.BlockSpec((D, D), lambda i: (0, 0)),
    ],
    out_specs=pl.BlockSpec((BR, D), lambda i: (i, 0)),
    out_shape=jax.ShapeDtypeStruct((N, D), jnp.float32),
)


def kernel(x, edge_index, W1, b1, W2, b2):
    src = edge_index[0]
    dst = edge_index[1]
    pad = E_PAD - E
    src_i = jnp.concatenate(
        [src, jnp.zeros((pad,), jnp.int32)]).reshape(NS * T, CHUNK)
    dst_i = jnp.concatenate(
        [dst, jnp.full((pad,), N, jnp.int32)]).reshape(NS * T, CHUNK)
    zb = jnp.zeros((NC, HD), jnp.float32)
    b1r = b1.reshape(1, D)
    b2s = b2.reshape(NC, HD)

    ax = _spmm(x, src_i, dst_i, zb)
    z2 = _dense(ax, W1, b1r, W2)
    return _spmm(z2, src_i, dst_i, b2s)
